# baseline (speedup 1.0000x reference)
import functools

import jax
import jax.numpy as jnp
from jax import lax
from jax.experimental import pallas as pl
from jax.experimental.pallas import tpu as pltpu
from jax.experimental.pallas import tpu_sc as plsc

N = 10000
E = 320000
D = 128
L = 16
P = 4
DH = D // 2
NPAD = 10240
ROWS_PER_TILE = NPAD // 16
CHUNK = 128
NBUF = 2
CHUNKS_PER_TILE = 160
EPAD = CHUNKS_PER_TILE * 16 * CHUNK


def _sc_body(x_hbm, lab_hbm, e2_hbm, w_hbm, b_hbm, out_hbm,
             labels_v, wtab_v, eidx_v, gidx_v, didx_v, wbuf_v, rows_v,
             tmp_i_v, acc_sh,
             sem, sem_g0, sem_g1, sem_s0, sem_s1):
    c = lax.axis_index("c")
    s = lax.axis_index("s")

    def run():
        coff = c * N
        pltpu.sync_copy(lab_hbm, labels_v)
        pltpu.sync_copy(w_hbm, wtab_v)
        pltpu.sync_copy(e2_hbm.at[pl.ds(s * CHUNKS_PER_TILE,
                                        CHUNKS_PER_TILE)], eidx_v)

        row0 = s * ROWS_PER_TILE
        for i in range(ROWS_PER_TILE // CHUNK):
            r = row0 + i * CHUNK
            pltpu.sync_copy(lab_hbm.at[pl.ds(r, CHUNK)], tmp_i_v)
            for g in range(CHUNK // 16):
                sl = pl.ds(g * 16, 16)
                tmp_i_v[sl] = tmp_i_v[sl] + c * L
            pltpu.async_copy(b_hbm.at[tmp_i_v], rows_v.at[0], sem).wait()
            pltpu.sync_copy(rows_v.at[0], acc_sh.at[pl.ds(r, CHUNK)])
        plsc.subcore_barrier()

        sem_g = (sem_g0, sem_g1)
        sem_s = (sem_s0, sem_s1)

        def scatter_wait(b):
            pltpu.make_async_copy(
                rows_v.at[b], acc_sh.at[didx_v.at[b]], sem_s[b]).wait()

        def stage(k, b):
            for g in range(CHUNK // 16):
                sl = pl.ds(g * 16, 16)
                gidx_v[b, sl] = eidx_v[k, 0, sl] + coff
            pltpu.async_copy(x_hbm.at[gidx_v.at[b]], rows_v.at[b], sem_g[b])

        def process(k, b):
            for g in range(CHUNK // 16):
                sl = pl.ds(g * 16, 16)
                s16 = eidx_v[k, 0, sl]
                dp16 = eidx_v[k, 1, sl]
                d16 = dp16 & 0x3FFF
                p16 = dp16 >> 14
                didx_v[b, sl] = d16
                ls = plsc.load_gather(labels_v, [s16])
                ld = plsc.load_gather(labels_v, [d16])
                widx = (ld * L + ls) * P + p16
                wbuf_v[b, sl] = plsc.load_gather(wtab_v, [widx])
            pltpu.make_async_copy(
                x_hbm.at[gidx_v.at[b]], rows_v.at[b], sem_g[b]).wait()

            def sgroup(g, cc):
                wv = wbuf_v[b, pl.ds(g * 16, 16)]
                for e in range(16):
                    w = wv[e]
                    row = g * 16 + e
                    for j in range(DH // 16):
                        jl = pl.ds(j * 16, 16)
                        rows_v[b, row, jl] = rows_v[b, row, jl] * w
                return cc
            lax.fori_loop(0, CHUNK // 16, sgroup, 0, unroll=2)
            pltpu.async_copy(rows_v.at[b], acc_sh.at[didx_v.at[b]],
                             sem_s[b], add=True)

        stage(0, 0)

        def chunk_body(ko, carry):
            for b in range(NBUF):
                k = ko * NBUF + b

                @pl.when((k >= 1) & (k + 1 < CHUNKS_PER_TILE))
                def _():
                    scatter_wait(1 - b)

                @pl.when(k + 1 < CHUNKS_PER_TILE)
                def _():
                    stage(k + 1, 1 - b)
                process(k, b)
            return carry
        lax.fori_loop(0, CHUNKS_PER_TILE // NBUF, chunk_body, 0)
        for q in range(NBUF):
            scatter_wait(q)
        plsc.subcore_barrier()

        for i in range(ROWS_PER_TILE // CHUNK):
            r = row0 + i * CHUNK
            pltpu.sync_copy(acc_sh.at[pl.ds(r, CHUNK)], rows_v.at[0])
            pltpu.sync_copy(rows_v.at[0],
                            out_hbm.at[pl.ds(c * NPAD + r, CHUNK)])

    run()


@jax.jit
def _run(xs, labels_pad, e3, wtab, bias_flat):
    mesh = plsc.VectorSubcoreMesh(core_axis_name="c", subcore_axis_name="s")
    kfn = pl.kernel(
        _sc_body,
        out_type=jax.ShapeDtypeStruct((2 * NPAD, DH), jnp.float32),
        mesh=mesh,
        compiler_params=pltpu.CompilerParams(
            needs_layout_passes=False, use_tc_tiling_on_sc=False),
        scratch_types=[
            pltpu.VMEM((NPAD,), jnp.int32),
            pltpu.VMEM((WTAB_PAD,), jnp.float32),
            pltpu.VMEM((CHUNKS_PER_TILE, 2, CHUNK), jnp.int32),
            pltpu.VMEM((NBUF, CHUNK), jnp.int32),
            pltpu.VMEM((NBUF, CHUNK), jnp.int32),
            pltpu.VMEM((NBUF, CHUNK), jnp.float32),
            pltpu.VMEM((NBUF, CHUNK, DH), jnp.float32),
            pltpu.VMEM((CHUNK,), jnp.int32),
            pltpu.VMEM_SHARED((NPAD, DH), jnp.float32),
        ] + [pltpu.SemaphoreType.DMA] * 5,
    )
    return kfn(xs, labels_pad, e3, wtab, bias_flat)


WTAB_PAD = L * L * P


def kernel(x, edge_index, node_labels, edge_property, Param_W, Param_b):
    xs = x.reshape(N, 2, DH).transpose(1, 0, 2).reshape(2 * N, DH)
    labels_pad = jnp.concatenate(
        [node_labels, jnp.zeros((NPAD - N,), jnp.int32)])
    src = jnp.concatenate(
        [edge_index[0], jnp.zeros((EPAD - E,), jnp.int32)])
    dst = jnp.concatenate(
        [edge_index[1], jnp.full((EPAD - E,), NPAD - 1, jnp.int32)])
    prop = jnp.concatenate(
        [edge_property, jnp.zeros((EPAD - E,), jnp.int32)])
    packed = dst + prop * 16384
    e2 = jnp.stack([src, packed]).reshape(2, EPAD // CHUNK, CHUNK)
    e2 = e2.transpose(1, 0, 2)
    bias_flat = Param_b.reshape(L, 2, DH).transpose(1, 0, 2).reshape(2 * L, DH)

    out2 = _run(xs, labels_pad, e2, Param_W, bias_flat)
    out2 = out2.reshape(2, NPAD, DH)[:, :N]
    return out2.transpose(1, 0, 2).reshape(N, D)


# speedup vs baseline: 1.2906x; 1.2906x over previous
import functools

import jax
import jax.numpy as jnp
from jax import lax
from jax.experimental import pallas as pl
from jax.experimental.pallas import tpu as pltpu
from jax.experimental.pallas import tpu_sc as plsc

N = 10000
E = 320000
D = 128
L = 16
P = 4
DH = D // 2
NPAD = 10240
ROWS_PER_TILE = NPAD // 16
CHUNK = 128
CHUNKS_PER_TILE = 158
EPAD = CHUNKS_PER_TILE * 16 * CHUNK


def _sc_body(x_hbm, lab_hbm, e3_hbm, w_hbm, b_hbm, out_hbm,
             labels_v, wtab_v, eidx_v, gidx_v, rows_v, tmp_i_v, acc_sh,
             sem, sem_g0, sem_g1, sem_h0, sem_h1, sem_s0, sem_s1):
    c = lax.axis_index("c")
    s = lax.axis_index("s")

    def run():
        coff = c * N
        pltpu.sync_copy(lab_hbm, labels_v)
        pltpu.sync_copy(w_hbm, wtab_v)
        pltpu.sync_copy(e3_hbm.at[pl.ds(s * CHUNKS_PER_TILE,
                                        CHUNKS_PER_TILE)], eidx_v)

        row0 = s * ROWS_PER_TILE
        for i in range(ROWS_PER_TILE // CHUNK):
            r = row0 + i * CHUNK
            pltpu.sync_copy(lab_hbm.at[pl.ds(r, CHUNK)], tmp_i_v)
            for g in range(CHUNK // 16):
                sl = pl.ds(g * 16, 16)
                tmp_i_v[sl] = tmp_i_v[sl] + c * L
            pltpu.async_copy(b_hbm.at[tmp_i_v], rows_v.at[0], sem).wait()
            pltpu.sync_copy(rows_v.at[0], acc_sh.at[pl.ds(r, CHUNK)])
        plsc.subcore_barrier()

        sem_g = (sem_g0, sem_g1)
        sem_h = (sem_h0, sem_h1)
        sem_s = (sem_s0, sem_s1)
        HALF = CHUNK // 2

        def scatter_wait(b):
            pltpu.make_async_copy(
                rows_v.at[b], acc_sh.at[eidx_v.at[0, 1]], sem_s[b]).wait()

        def stage(k, b):
            for g in range(CHUNK // 16):
                sl = pl.ds(g * 16, 16)
                gidx_v[b, g // 4, pl.ds((g % 4) * 16, 16)] = (
                    eidx_v[k, 0, sl] + coff)
            pltpu.async_copy(x_hbm.at[gidx_v.at[b, 0]],
                             rows_v.at[b, pl.ds(0, HALF)], sem_g[b])
            pltpu.async_copy(x_hbm.at[gidx_v.at[b, 1]],
                             rows_v.at[b, pl.ds(HALF, HALF)], sem_h[b])

        def process(k, b):
            wvs = []
            for g in range(CHUNK // 16):
                sl = pl.ds(g * 16, 16)
                s16 = eidx_v[k, 0, sl]
                d16 = eidx_v[k, 1, sl]
                p16 = eidx_v[k, 2, sl]
                ls = plsc.load_gather(labels_v, [s16])
                ld = plsc.load_gather(labels_v, [d16])
                widx = (ld * L + ls) * P + p16
                wvs.append(plsc.load_gather(wtab_v, [widx]))
            pltpu.make_async_copy(
                x_hbm.at[gidx_v.at[b, 0]],
                rows_v.at[b, pl.ds(0, HALF)], sem_g[b]).wait()
            for g in range(CHUNK // 16):
                if g == 4:
                    pltpu.make_async_copy(
                        x_hbm.at[gidx_v.at[b, 1]],
                        rows_v.at[b, pl.ds(HALF, HALF)], sem_h[b]).wait()
                wv = wvs[g]
                for e in range(16):
                    w = wv[e]
                    row = g * 16 + e
                    for j in range(DH // 16):
                        jl = pl.ds(j * 16, 16)
                        rows_v[b, row, jl] = rows_v[b, row, jl] * w
            pltpu.async_copy(rows_v.at[b], acc_sh.at[eidx_v.at[k, 1]],
                             sem_s[b], add=True)

        stage(0, 0)

        def chunk_body(ko, carry):
            for b in range(2):
                k = ko * 2 + b

                @pl.when((k >= 1) & (k + 1 < CHUNKS_PER_TILE))
                def _():
                    scatter_wait(1 - b)

                @pl.when(k + 1 < CHUNKS_PER_TILE)
                def _():
                    stage(k + 1, 1 - b)
                process(k, b)
            return carry
        lax.fori_loop(0, CHUNKS_PER_TILE // 2, chunk_body, 0)
        scatter_wait(0)
        scatter_wait(1)
        plsc.subcore_barrier()

        for i in range(ROWS_PER_TILE // CHUNK):
            r = row0 + i * CHUNK
            pltpu.sync_copy(acc_sh.at[pl.ds(r, CHUNK)], rows_v.at[0])
            pltpu.sync_copy(rows_v.at[0],
                            out_hbm.at[pl.ds(c * NPAD + r, CHUNK)])

    run()


@jax.jit
def _run(xs, labels_pad, e3, wtab, bias_flat):
    mesh = plsc.VectorSubcoreMesh(core_axis_name="c", subcore_axis_name="s")
    kfn = pl.kernel(
        _sc_body,
        out_type=jax.ShapeDtypeStruct((2 * NPAD, DH), jnp.float32),
        mesh=mesh,
        compiler_params=pltpu.CompilerParams(
            needs_layout_passes=False, use_tc_tiling_on_sc=False),
        scratch_types=[
            pltpu.VMEM((NPAD,), jnp.int32),
            pltpu.VMEM((WTAB_PAD,), jnp.float32),
            pltpu.VMEM((CHUNKS_PER_TILE, 3, CHUNK), jnp.int32),
            pltpu.VMEM((2, 2, CHUNK // 2), jnp.int32),
            pltpu.VMEM((2, CHUNK, DH), jnp.float32),
            pltpu.VMEM((CHUNK,), jnp.int32),
            pltpu.VMEM_SHARED((NPAD, DH), jnp.float32),
        ] + [pltpu.SemaphoreType.DMA] * 7,
    )
    return kfn(xs, labels_pad, e3, wtab, bias_flat)


WTAB_PAD = L * L * P


def kernel(x, edge_index, node_labels, edge_property, Param_W, Param_b):
    xs = x.reshape(N, 2, DH).transpose(1, 0, 2).reshape(2 * N, DH)
    labels_pad = jnp.concatenate(
        [node_labels, jnp.zeros((NPAD - N,), jnp.int32)])
    src = jnp.concatenate(
        [edge_index[0], jnp.zeros((EPAD - E,), jnp.int32)])
    dst = jnp.concatenate(
        [edge_index[1], jnp.full((EPAD - E,), NPAD - 1, jnp.int32)])
    prop = jnp.concatenate(
        [edge_property, jnp.zeros((EPAD - E,), jnp.int32)])
    e3 = jnp.stack([src, dst, prop]).reshape(3, EPAD // CHUNK, CHUNK)
    e3 = e3.transpose(1, 0, 2)
    bias_flat = Param_b.reshape(L, 2, DH).transpose(1, 0, 2).reshape(2 * L, DH)

    out2 = _run(xs, labels_pad, e3, Param_W, bias_flat)
    out2 = out2.reshape(2, NPAD, DH)[:, :N]
    return out2.transpose(1, 0, 2).reshape(N, D)


# speedup vs baseline: 1.2933x; 1.0021x over previous
import jax
import jax.numpy as jnp
from jax import lax
from jax.experimental import pallas as pl
from jax.experimental.pallas import tpu as pltpu
from jax.experimental.pallas import tpu_sc as plsc

N = 10000
E = 320000
D = 128
L = 16
P = 4
DH = D // 2
NPAD = 10240
ROWS_PER_TILE = NPAD // 16
CHUNK = 128
CHUNKS_PER_TILE = 158
EPAD = CHUNKS_PER_TILE * 16 * CHUNK


def _sc_body(x_hbm, lab_hbm, e3_hbm, w_hbm, b_hbm, out_hbm,
             labels_v, wtab_v, eidx_v, gidx_v, rows_v, tmp_i_v, acc_sh,
             sem, sem_g0, sem_g1, sem_h0, sem_h1, sem_s0, sem_s1):
    c = lax.axis_index("c")
    s = lax.axis_index("s")

    def run():
        coff = c * N
        pltpu.sync_copy(lab_hbm, labels_v)
        pltpu.sync_copy(w_hbm, wtab_v)
        pltpu.sync_copy(e3_hbm.at[pl.ds(s * CHUNKS_PER_TILE,
                                        CHUNKS_PER_TILE)], eidx_v)

        row0 = s * ROWS_PER_TILE
        for i in range(ROWS_PER_TILE // CHUNK):
            r = row0 + i * CHUNK
            pltpu.sync_copy(lab_hbm.at[pl.ds(r, CHUNK)], tmp_i_v)
            for g in range(CHUNK // 16):
                sl = pl.ds(g * 16, 16)
                tmp_i_v[sl] = tmp_i_v[sl] + c * L
            pltpu.async_copy(b_hbm.at[tmp_i_v], rows_v.at[0], sem).wait()
            pltpu.sync_copy(rows_v.at[0], acc_sh.at[pl.ds(r, CHUNK)])
        plsc.subcore_barrier()

        sem_g = (sem_g0, sem_g1)
        sem_h = (sem_h0, sem_h1)
        sem_s = (sem_s0, sem_s1)
        HALF = CHUNK // 2

        def scatter_wait(b):
            pltpu.make_async_copy(
                rows_v.at[b], acc_sh.at[eidx_v.at[0, 1]], sem_s[b]).wait()

        def stage(k, b):
            for g in range(CHUNK // 16):
                sl = pl.ds(g * 16, 16)
                gidx_v[b, g // 4, pl.ds((g % 4) * 16, 16)] = (
                    eidx_v[k, 0, sl] + coff)
            pltpu.async_copy(x_hbm.at[gidx_v.at[b, 0]],
                             rows_v.at[b, pl.ds(0, HALF)], sem_g[b])
            pltpu.async_copy(x_hbm.at[gidx_v.at[b, 1]],
                             rows_v.at[b, pl.ds(HALF, HALF)], sem_h[b])

        def process(k, b):
            wvs = []
            for g in range(CHUNK // 16):
                sl = pl.ds(g * 16, 16)
                s16 = eidx_v[k, 0, sl]
                d16 = eidx_v[k, 1, sl]
                p16 = eidx_v[k, 2, sl]
                ls = plsc.load_gather(labels_v, [s16])
                ld = plsc.load_gather(labels_v, [d16])
                widx = (ld * L + ls) * P + p16
                wvs.append(plsc.load_gather(wtab_v, [widx]))
            pltpu.make_async_copy(
                x_hbm.at[gidx_v.at[b, 0]],
                rows_v.at[b, pl.ds(0, HALF)], sem_g[b]).wait()
            for g in range(CHUNK // 16):
                if g == 4:
                    pltpu.make_async_copy(
                        x_hbm.at[gidx_v.at[b, 1]],
                        rows_v.at[b, pl.ds(HALF, HALF)], sem_h[b]).wait()
                wv = wvs[g]
                for e in range(16):
                    w = wv[e]
                    row = g * 16 + e
                    for j in range(DH // 16):
                        jl = pl.ds(j * 16, 16)
                        rows_v[b, row, jl] = rows_v[b, row, jl] * w
            pltpu.async_copy(rows_v.at[b], acc_sh.at[eidx_v.at[k, 1]],
                             sem_s[b], add=True)

        stage(0, 0)

        def chunk_body(ko, carry):
            for b in range(2):
                k = ko * 2 + b

                @pl.when((k >= 1) & (k + 1 < CHUNKS_PER_TILE))
                def _():
                    scatter_wait(1 - b)

                @pl.when(k + 1 < CHUNKS_PER_TILE)
                def _():
                    stage(k + 1, 1 - b)
                process(k, b)
            return carry
        lax.fori_loop(0, CHUNKS_PER_TILE // 2, chunk_body, 0)
        scatter_wait(0)
        scatter_wait(1)
        plsc.subcore_barrier()

        for i in range(ROWS_PER_TILE // CHUNK):
            r = row0 + i * CHUNK
            pltpu.sync_copy(acc_sh.at[pl.ds(r, CHUNK)], rows_v.at[0])
            pltpu.sync_copy(rows_v.at[0],
                            out_hbm.at[pl.ds(c * NPAD + r, CHUNK)])

    run()


@jax.jit
def _run(xs, labels_pad, e3, wtab, bias_flat):
    mesh = plsc.VectorSubcoreMesh(core_axis_name="c", subcore_axis_name="s")
    kfn = pl.kernel(
        _sc_body,
        out_type=jax.ShapeDtypeStruct((2 * NPAD, DH), jnp.float32),
        mesh=mesh,
        compiler_params=pltpu.CompilerParams(
            needs_layout_passes=False, use_tc_tiling_on_sc=False),
        scratch_types=[
            pltpu.VMEM((NPAD,), jnp.int32),
            pltpu.VMEM((WTAB_PAD,), jnp.float32),
            pltpu.VMEM((CHUNKS_PER_TILE, 3, CHUNK), jnp.int32),
            pltpu.VMEM((2, 2, CHUNK // 2), jnp.int32),
            pltpu.VMEM((2, CHUNK, DH), jnp.float32),
            pltpu.VMEM((CHUNK,), jnp.int32),
            pltpu.VMEM_SHARED((NPAD, DH), jnp.float32),
        ] + [pltpu.SemaphoreType.DMA] * 7,
    )
    return kfn(xs, labels_pad, e3, wtab, bias_flat)


WTAB_PAD = L * L * P


def kernel(x, edge_index, node_labels, edge_property, Param_W, Param_b):
    xs = x.reshape(N, 2, DH).transpose(1, 0, 2).reshape(2 * N, DH)
    labels_pad = jnp.concatenate(
        [node_labels, jnp.zeros((NPAD - N,), jnp.int32)])
    src = jnp.concatenate(
        [edge_index[0], jnp.zeros((EPAD - E,), jnp.int32)])
    dst = jnp.concatenate(
        [edge_index[1], jnp.full((EPAD - E,), NPAD - 1, jnp.int32)])
    prop = jnp.concatenate(
        [edge_property, jnp.zeros((EPAD - E,), jnp.int32)])
    e3 = jnp.stack([src, dst, prop]).reshape(3, EPAD // CHUNK, CHUNK)
    e3 = e3.transpose(1, 0, 2)
    bias_flat = Param_b.reshape(L, 2, DH).transpose(1, 0, 2).reshape(2 * L, DH)

    out2 = _run(xs, labels_pad, e3, Param_W, bias_flat)
    out2 = out2.reshape(2, NPAD, DH)[:, :N]
    return out2.transpose(1, 0, 2).reshape(N, D)
